# baseline (speedup 1.0000x reference)
import jax
import jax.numpy as jnp
from jax import lax
from jax.experimental import pallas as pl
from jax.experimental.pallas import tpu as pltpu
from jax.experimental.pallas import tpu_sc as plsc

_B, _S, _G, _F = 4096, 50, 1, 2
_D = 32
_FIELD_OFFSET = 1000000
_NUM_ROWS = 2000000

_M = _B * _S * _G
_N = _M * _F
_NC, _NS = 2, 16
_NW = _NC * _NS
_TPW = _M // _NW
_CHUNK = 128
_JPW = _TPW // _CHUNK


def _gather_body(x1_hbm, table_hbm, emb0_hbm, emb1_hbm,
                 pos0_v, pos1_v, x0_v, x1_v,
                 r00, r01, r10, r11, sema, semb, sem0, sem1):
    wid = lax.axis_index("s") * _NC + lax.axis_index("c")
    t0 = wid * _TPW
    lane = lax.iota(jnp.int32, 16)

    def mk_pos(j, carry):
        for k in range(_CHUNK // 16):
            p0 = (t0 + j * _CHUNK + k * 16 + lane) * 2
            pos0_v[j, pl.ds(k * 16, 16)] = p0
            pos1_v[j, pl.ds(k * 16, 16)] = p0 + 1
        return carry

    lax.fori_loop(0, _JPW, mk_pos, 0)

    ha = [pltpu.async_copy(x1_hbm.at[pos0_v.at[j]], x0_v.at[j], sema)
          for j in range(_JPW)]
    hb = [pltpu.async_copy(x1_hbm.at[pos1_v.at[j]], x1_v.at[j], semb)
          for j in range(_JPW)]
    for h in ha + hb:
        h.wait()

    table1 = table_hbm.at[pl.ds(_FIELD_OFFSET, _NUM_ROWS - _FIELD_OFFSET)]

    def fetch(j, idx_v, rows, sem, src, emb_hbm):
        pltpu.async_copy(src.at[idx_v.at[j]], rows, sem).wait()
        pltpu.sync_copy(rows, emb_hbm.at[pl.ds((t0 + j * _CHUNK), _CHUNK)])

    def chunk(i, carry):
        fetch(2 * i, x0_v, r00, sem0, table_hbm, emb0_hbm)
        fetch(2 * i, x1_v, r10, sem1, table1, emb1_hbm)
        fetch(2 * i + 1, x0_v, r01, sem0, table_hbm, emb0_hbm)
        fetch(2 * i + 1, x1_v, r11, sem1, table1, emb1_hbm)
        return carry

    lax.fori_loop(0, _JPW // 2, chunk, 0)


_gather = pl.kernel(
    _gather_body,
    out_type=(
        jax.ShapeDtypeStruct((_M, _D), jnp.float32),
        jax.ShapeDtypeStruct((_M, _D), jnp.float32),
    ),
    mesh=plsc.VectorSubcoreMesh(core_axis_name="c", subcore_axis_name="s"),
    compiler_params=pltpu.CompilerParams(use_tc_tiling_on_sc=False),
    scratch_types=[
        pltpu.VMEM((_JPW, _CHUNK), jnp.int32),
        pltpu.VMEM((_JPW, _CHUNK), jnp.int32),
        pltpu.VMEM((_JPW, _CHUNK), jnp.int32),
        pltpu.VMEM((_JPW, _CHUNK), jnp.int32),
        pltpu.VMEM((_CHUNK, _D), jnp.float32),
        pltpu.VMEM((_CHUNK, _D), jnp.float32),
        pltpu.VMEM((_CHUNK, _D), jnp.float32),
        pltpu.VMEM((_CHUNK, _D), jnp.float32),
        pltpu.SemaphoreType.DMA,
        pltpu.SemaphoreType.DMA,
        pltpu.SemaphoreType.DMA,
        pltpu.SemaphoreType.DMA,
    ],
)


def _proj_body(e0_ref, e1_ref, w0_ref, w1_ref, b_ref, out_ref):
    dn = (((1,), (1,)), ((), ()))
    out_ref[...] = (
        lax.dot_general(e0_ref[...], w0_ref[...], dn,
                        preferred_element_type=jnp.float32)
        + lax.dot_general(e1_ref[...], w1_ref[...], dn,
                          preferred_element_type=jnp.float32)
        + b_ref[...]
    )


_BLK = 2048


def _proj(e0, e1, w0, w1, b2):
    return pl.pallas_call(
        _proj_body,
        grid=(_M // _BLK,),
        in_specs=[
            pl.BlockSpec((_BLK, _D), lambda i: (i, 0)),
            pl.BlockSpec((_BLK, _D), lambda i: (i, 0)),
            pl.BlockSpec((_D, _D), lambda i: (0, 0)),
            pl.BlockSpec((_D, _D), lambda i: (0, 0)),
            pl.BlockSpec((1, _D), lambda i: (0, 0)),
        ],
        out_specs=pl.BlockSpec((_BLK, _D), lambda i: (i, 0)),
        out_shape=jax.ShapeDtypeStruct((_M, _D), jnp.float32),
    )(e0, e1, w0, w1, b2)


def kernel(x, table, W, b):
    emb0, emb1 = _gather(x.reshape(_N), table)
    out = _proj(emb0, emb1, W[:, :_D], W[:, _D:], b.reshape(1, _D))
    return out.reshape(_B, _S, _G, _D)


# speedup vs baseline: 1.0626x; 1.0626x over previous
import jax
import jax.numpy as jnp
from jax import lax
from jax.experimental import pallas as pl
from jax.experimental.pallas import tpu as pltpu
from jax.experimental.pallas import tpu_sc as plsc

_B, _S, _G, _F = 4096, 50, 1, 2
_D = 32
_FIELD_OFFSET = 1000000

_N = _B * _S * _G * _F
_CHUNK = 128
_NROWS = _N // _CHUNK
_NC, _NS = 2, 16
_NW = _NC * _NS
_RPW = _NROWS // _NW


def _gather_body(idx_hbm, table_hbm, out_hbm, idx_v, rows0, rows1, sem0, sem1):
    wid = lax.axis_index("s") * _NC + lax.axis_index("c")
    base = wid * _RPW
    pltpu.sync_copy(idx_hbm.at[wid], idx_v)

    offs = (lax.iota(jnp.int32, 16) % 2) * _FIELD_OFFSET

    def add_offs(j, carry):
        for k in range(_CHUNK // 16):
            sl = pl.ds(k * 16, 16)
            idx_v[j, sl] = idx_v[j, sl] + offs
        return carry

    lax.fori_loop(0, _RPW, add_offs, 0)

    def fetch(j, rows, sem):
        pltpu.async_copy(table_hbm.at[idx_v.at[j]], rows, sem).wait()
        pltpu.sync_copy(rows, out_hbm.at[pl.ds((base + j) * _CHUNK, _CHUNK)])

    def chunk(i, carry):
        fetch(2 * i, rows0, sem0)
        fetch(2 * i + 1, rows1, sem1)
        return carry

    lax.fori_loop(0, _RPW // 2, chunk, 0)


_gather = pl.kernel(
    _gather_body,
    out_type=jax.ShapeDtypeStruct((_N, _D), jnp.float32),
    mesh=plsc.VectorSubcoreMesh(core_axis_name="c", subcore_axis_name="s"),
    compiler_params=pltpu.CompilerParams(use_tc_tiling_on_sc=False),
    scratch_types=[
        pltpu.VMEM((_RPW, _CHUNK), jnp.int32),
        pltpu.VMEM((_CHUNK, _D), jnp.float32),
        pltpu.VMEM((_CHUNK, _D), jnp.float32),
        pltpu.SemaphoreType.DMA,
        pltpu.SemaphoreType.DMA,
    ],
)


def _proj_body(emb_ref, w_ref, b_ref, out_ref):
    out_ref[...] = lax.dot_general(
        emb_ref[...], w_ref[...],
        (((1,), (1,)), ((), ())),
        preferred_element_type=jnp.float32,
    ) + b_ref[...]


_M = _N // _F
_BLK = 2048


def _proj(emb, w, b2):
    return pl.pallas_call(
        _proj_body,
        grid=(_M // _BLK,),
        in_specs=[
            pl.BlockSpec((_BLK, _D * _F), lambda i: (i, 0)),
            pl.BlockSpec((_D, _D * _F), lambda i: (0, 0)),
            pl.BlockSpec((1, _D), lambda i: (0, 0)),
        ],
        out_specs=pl.BlockSpec((_BLK, _D), lambda i: (i, 0)),
        out_shape=jax.ShapeDtypeStruct((_M, _D), jnp.float32),
    )(emb, w, b2)


def kernel(x, table, W, b):
    idx = x.reshape(_NW, _RPW, _CHUNK)
    emb = _gather(idx, table)
    out = _proj(emb.reshape(_M, _D * _F), W, b.reshape(1, _D))
    return out.reshape(_B, _S, _G, _D)


# speedup vs baseline: 1.0853x; 1.0214x over previous
import jax
import jax.numpy as jnp
from jax import lax
from jax.experimental import pallas as pl
from jax.experimental.pallas import tpu as pltpu
from jax.experimental.pallas import tpu_sc as plsc

_B, _S, _G, _F = 4096, 50, 1, 2
_D = 32
_FIELD_OFFSET = 1000000

_N = _B * _S * _G * _F
_CHUNK = 128
_NROWS = _N // _CHUNK
_NC, _NS = 2, 16
_NW = _NC * _NS
_RPW = _NROWS // _NW


def _gather_body(idx_hbm, table_hbm, out_hbm, idx_v,
                 rows0, rows1, rows2, rows3, sem0, sem1, sem2, sem3):
    wid = lax.axis_index("s") * _NC + lax.axis_index("c")
    base = wid * _RPW
    pltpu.sync_copy(idx_hbm.at[wid], idx_v)

    offs = (lax.iota(jnp.int32, 16) % 2) * _FIELD_OFFSET

    def add_offs(j, carry):
        for k in range(_CHUNK // 16):
            sl = pl.ds(k * 16, 16)
            idx_v[j, sl] = idx_v[j, sl] + offs
        return carry

    lax.fori_loop(0, _RPW, add_offs, 0)

    def fetch(j, rows, sem):
        pltpu.async_copy(table_hbm.at[idx_v.at[j]], rows, sem).wait()
        pltpu.sync_copy(rows, out_hbm.at[pl.ds((base + j) * _CHUNK, _CHUNK)])

    def chunk(i, carry):
        fetch(4 * i, rows0, sem0)
        fetch(4 * i + 1, rows1, sem1)
        fetch(4 * i + 2, rows2, sem2)
        fetch(4 * i + 3, rows3, sem3)
        return carry

    lax.fori_loop(0, _RPW // 4, chunk, 0)


_gather = pl.kernel(
    _gather_body,
    out_type=jax.ShapeDtypeStruct((_N, _D), jnp.float32),
    mesh=plsc.VectorSubcoreMesh(core_axis_name="c", subcore_axis_name="s"),
    compiler_params=pltpu.CompilerParams(use_tc_tiling_on_sc=False),
    scratch_types=[
        pltpu.VMEM((_RPW, _CHUNK), jnp.int32),
        pltpu.VMEM((_CHUNK, _D), jnp.float32),
        pltpu.VMEM((_CHUNK, _D), jnp.float32),
        pltpu.VMEM((_CHUNK, _D), jnp.float32),
        pltpu.VMEM((_CHUNK, _D), jnp.float32),
        pltpu.SemaphoreType.DMA,
        pltpu.SemaphoreType.DMA,
        pltpu.SemaphoreType.DMA,
        pltpu.SemaphoreType.DMA,
    ],
)


def _proj_body(emb_ref, w_ref, b_ref, out_ref):
    out_ref[...] = lax.dot_general(
        emb_ref[...], w_ref[...],
        (((1,), (1,)), ((), ())),
        preferred_element_type=jnp.float32,
    ) + b_ref[...]


_M = _N // _F
_BLK = 4096


def _proj(emb, w, b2):
    return pl.pallas_call(
        _proj_body,
        grid=(_M // _BLK,),
        in_specs=[
            pl.BlockSpec((_BLK, _D * _F), lambda i: (i, 0)),
            pl.BlockSpec((_D, _D * _F), lambda i: (0, 0)),
            pl.BlockSpec((1, _D), lambda i: (0, 0)),
        ],
        out_specs=pl.BlockSpec((_BLK, _D), lambda i: (i, 0)),
        out_shape=jax.ShapeDtypeStruct((_M, _D), jnp.float32),
    )(emb, w, b2)


def kernel(x, table, W, b):
    idx = x.reshape(_NW, _RPW, _CHUNK)
    emb = _gather(idx, table)
    out = _proj(emb.reshape(_M, _D * _F), W, b.reshape(1, _D))
    return out.reshape(_B, _S, _G, _D)


# speedup vs baseline: 1.0924x; 1.0066x over previous
import jax
import jax.numpy as jnp
from jax import lax
from jax.experimental import pallas as pl
from jax.experimental.pallas import tpu as pltpu
from jax.experimental.pallas import tpu_sc as plsc

_B, _S, _G, _F = 4096, 50, 1, 2
_D = 32
_FIELD_OFFSET = 1000000

_N = _B * _S * _G * _F
_CHUNK = 128
_NROWS = _N // _CHUNK
_NC, _NS = 2, 16
_NW = _NC * _NS
_RPW = _NROWS // _NW


def _gather_body(idx_hbm, table_hbm, out_hbm, idx_v,
                 rows0, rows1, rows2, rows3, sem0, sem1, sem2, sem3):
    wid = lax.axis_index("s") * _NC + lax.axis_index("c")
    base = wid * _RPW
    pltpu.sync_copy(idx_hbm.at[wid], idx_v)

    offs = (lax.iota(jnp.int32, 16) % 2) * _FIELD_OFFSET

    def add_offs(j, carry):
        for k in range(_CHUNK // 16):
            sl = pl.ds(k * 16, 16)
            idx_v[j, sl] = idx_v[j, sl] + offs
        return carry

    lax.fori_loop(0, _RPW, add_offs, 0)

    def fetch(j, rows, sem):
        pltpu.async_copy(table_hbm.at[idx_v.at[j]], rows, sem).wait()
        pltpu.sync_copy(rows, out_hbm.at[pl.ds((base + j) * _CHUNK, _CHUNK)])

    def chunk(i, carry):
        fetch(4 * i, rows0, sem0)
        fetch(4 * i + 1, rows1, sem1)
        fetch(4 * i + 2, rows2, sem2)
        fetch(4 * i + 3, rows3, sem3)
        return carry

    lax.fori_loop(0, _RPW // 4, chunk, 0)


_gather = pl.kernel(
    _gather_body,
    out_type=jax.ShapeDtypeStruct((_N, _D), jnp.float32),
    mesh=plsc.VectorSubcoreMesh(core_axis_name="c", subcore_axis_name="s"),
    compiler_params=pltpu.CompilerParams(use_tc_tiling_on_sc=False),
    scratch_types=[
        pltpu.VMEM((_RPW, _CHUNK), jnp.int32),
        pltpu.VMEM((_CHUNK, _D), jnp.float32),
        pltpu.VMEM((_CHUNK, _D), jnp.float32),
        pltpu.VMEM((_CHUNK, _D), jnp.float32),
        pltpu.VMEM((_CHUNK, _D), jnp.float32),
        pltpu.SemaphoreType.DMA,
        pltpu.SemaphoreType.DMA,
        pltpu.SemaphoreType.DMA,
        pltpu.SemaphoreType.DMA,
    ],
)


def _proj_body(emb_ref, w_ref, b_ref, out_ref):
    out_ref[...] = lax.dot_general(
        emb_ref[...], w_ref[...],
        (((1,), (1,)), ((), ())),
        preferred_element_type=jnp.float32,
    ) + b_ref[...]


_M = _N // _F
_BLK = 8192


def _proj(emb, w, b2):
    return pl.pallas_call(
        _proj_body,
        grid=(_M // _BLK,),
        in_specs=[
            pl.BlockSpec((_BLK, _D * _F), lambda i: (i, 0)),
            pl.BlockSpec((_D, _D * _F), lambda i: (0, 0)),
            pl.BlockSpec((1, _D), lambda i: (0, 0)),
        ],
        out_specs=pl.BlockSpec((_BLK, _D), lambda i: (i, 0)),
        out_shape=jax.ShapeDtypeStruct((_M, _D), jnp.float32),
    )(emb, w, b2)


def kernel(x, table, W, b):
    idx = x.reshape(_NW, _RPW, _CHUNK)
    emb = _gather(idx, table)
    out = _proj(emb.reshape(_M, _D * _F), W, b.reshape(1, _D))
    return out.reshape(_B, _S, _G, _D)


# speedup vs baseline: 1.0953x; 1.0026x over previous
import jax
import jax.numpy as jnp
from jax import lax
from jax.experimental import pallas as pl
from jax.experimental.pallas import tpu as pltpu
from jax.experimental.pallas import tpu_sc as plsc

_B, _S, _G, _F = 4096, 50, 1, 2
_D = 32
_FIELD_OFFSET = 1000000

_N = _B * _S * _G * _F
_CHUNK = 128
_NROWS = _N // _CHUNK
_NC, _NS = 2, 16
_NW = _NC * _NS
_RPW = _NROWS // _NW


def _gather_body(idx_hbm, table_hbm, out_hbm, idx_v,
                 rows0, rows1, rows2, rows3, sem0, sem1, sem2, sem3):
    wid = lax.axis_index("s") * _NC + lax.axis_index("c")
    base = wid * _RPW
    pltpu.sync_copy(idx_hbm.at[wid], idx_v)

    offs = (lax.iota(jnp.int32, 16) % 2) * _FIELD_OFFSET

    def add_offs(j, carry):
        for k in range(_CHUNK // 16):
            sl = pl.ds(k * 16, 16)
            idx_v[j, sl] = idx_v[j, sl] + offs
        return carry

    lax.fori_loop(0, _RPW, add_offs, 0)

    def fetch(j, rows, sem):
        pltpu.async_copy(table_hbm.at[idx_v.at[j]], rows, sem).wait()
        pltpu.sync_copy(rows, out_hbm.at[pl.ds((base + j) * _CHUNK, _CHUNK)])

    def chunk(i, carry):
        fetch(4 * i, rows0, sem0)
        fetch(4 * i + 1, rows1, sem1)
        fetch(4 * i + 2, rows2, sem2)
        fetch(4 * i + 3, rows3, sem3)
        return carry

    lax.fori_loop(0, _RPW // 4, chunk, 0)


_gather = pl.kernel(
    _gather_body,
    out_type=jax.ShapeDtypeStruct((_N, _D), jnp.float32),
    mesh=plsc.VectorSubcoreMesh(core_axis_name="c", subcore_axis_name="s"),
    compiler_params=pltpu.CompilerParams(use_tc_tiling_on_sc=False),
    scratch_types=[
        pltpu.VMEM((_RPW, _CHUNK), jnp.int32),
        pltpu.VMEM((_CHUNK, _D), jnp.float32),
        pltpu.VMEM((_CHUNK, _D), jnp.float32),
        pltpu.VMEM((_CHUNK, _D), jnp.float32),
        pltpu.VMEM((_CHUNK, _D), jnp.float32),
        pltpu.SemaphoreType.DMA,
        pltpu.SemaphoreType.DMA,
        pltpu.SemaphoreType.DMA,
        pltpu.SemaphoreType.DMA,
    ],
)


def _proj_body(emb_ref, w_ref, b_ref, out_ref):
    out_ref[...] = lax.dot_general(
        emb_ref[...], w_ref[...],
        (((1,), (1,)), ((), ())),
        preferred_element_type=jnp.float32,
    ) + b_ref[...]


_M = _N // _F
_BLK = 20480


def _proj(emb, w, b2):
    return pl.pallas_call(
        _proj_body,
        grid=(_M // _BLK,),
        in_specs=[
            pl.BlockSpec((_BLK, _D * _F), lambda i: (i, 0)),
            pl.BlockSpec((_D, _D * _F), lambda i: (0, 0)),
            pl.BlockSpec((1, _D), lambda i: (0, 0)),
        ],
        out_specs=pl.BlockSpec((_BLK, _D), lambda i: (i, 0)),
        out_shape=jax.ShapeDtypeStruct((_M, _D), jnp.float32),
    )(emb, w, b2)


def kernel(x, table, W, b):
    idx = x.reshape(_NW, _RPW, _CHUNK)
    emb = _gather(idx, table)
    out = _proj(emb.reshape(_M, _D * _F), W, b.reshape(1, _D))
    return out.reshape(_B, _S, _G, _D)
